# baseline (speedup 1.0000x reference)
import jax
import jax.numpy as jnp
from jax import lax
from jax.experimental import pallas as pl
from jax.experimental.pallas import tpu as pltpu
from jax.experimental.pallas import tpu_sc as plsc

_NUM_CORES = 2
_NUM_SUBCORES = 16
_NUM_WORKERS = _NUM_CORES * _NUM_SUBCORES
_LANES = 128
_PACK_ROWS = 4000


def _pack_body(t_ref, w_ref):
    w_ref[:, 0:64] = t_ref[...]


def _pack_wide(table):
    vocab, dim = table.shape
    return pl.pallas_call(
        _pack_body,
        grid=(vocab // _PACK_ROWS,),
        in_specs=[
            pl.BlockSpec((_PACK_ROWS, dim), lambda i: (i, 0)),
        ],
        out_specs=pl.BlockSpec((_PACK_ROWS, _LANES), lambda i: (i, 0)),
        out_shape=jax.ShapeDtypeStruct((vocab, _LANES), jnp.float32),
        compiler_params=pltpu.CompilerParams(
            dimension_semantics=("parallel",),
        ),
    )(table)


def _make_gather(batch: int, dim: int):
    assert batch % (8 * _NUM_WORKERS) == 0
    b_per_w = batch // _NUM_WORKERS

    mesh = plsc.VectorSubcoreMesh(core_axis_name="c", subcore_axis_name="s")

    chunk = b_per_w // 2

    def body(wide_hbm, idx_hbm, out_hbm, idx_v, rows_v, comp_v, sem):
        wid = lax.axis_index("s") * _NUM_CORES + lax.axis_index("c")
        base = wid * b_per_w
        pltpu.sync_copy(idx_hbm.at[pl.ds(base, b_per_w)], idx_v)

        @pl.loop(0, b_per_w, step=chunk)
        def _(c):
            pltpu.async_copy(
                wide_hbm.at[idx_v.at[pl.ds(c, chunk)]], rows_v, sem
            ).wait()

            @pl.loop(0, chunk)
            def _(j):
                for h in range(0, dim, 16):
                    comp_v[j, pl.ds(h, 16)] = rows_v[j, pl.ds(h, 16)]

            pltpu.sync_copy(comp_v, out_hbm.at[pl.ds(base + c, chunk)])

    return pl.kernel(
        body,
        mesh=mesh,
        out_type=jax.ShapeDtypeStruct((batch, dim), jnp.float32),
        scratch_types=[
            pltpu.VMEM((b_per_w,), jnp.int32),
            pltpu.VMEM((b_per_w // 2, _LANES), jnp.float32),
            pltpu.VMEM((b_per_w // 2, dim), jnp.float32),
            pltpu.SemaphoreType.DMA,
        ],
    )


@jax.jit
def kernel(indices, table):
    batch = indices.shape[0]
    dim = table.shape[1]
    idx = indices.astype(jnp.int32)
    wide = _pack_wide(table)
    return _make_gather(batch, dim)(wide, idx)


# speedup vs baseline: 1.1353x; 1.1353x over previous
import jax
import jax.numpy as jnp
from jax import lax
from jax.experimental import pallas as pl
from jax.experimental.pallas import tpu as pltpu
from jax.experimental.pallas import tpu_sc as plsc

_NUM_CORES = 2
_NUM_SUBCORES = 16
_NUM_WORKERS = _NUM_CORES * _NUM_SUBCORES


def _make_gather(batch: int, dim: int):
    assert batch % (8 * _NUM_WORKERS) == 0
    b_per_w = batch // _NUM_WORKERS

    mesh = plsc.VectorSubcoreMesh(core_axis_name="c", subcore_axis_name="s")

    def body(table_hbm, idx_hbm, out_hbm, idx_v, rows_v, sem):
        wid = lax.axis_index("s") * _NUM_CORES + lax.axis_index("c")
        base = wid * b_per_w
        pltpu.sync_copy(idx_hbm.at[pl.ds(base, b_per_w)], idx_v)
        pltpu.async_copy(table_hbm.at[idx_v], rows_v, sem).wait()
        pltpu.sync_copy(rows_v, out_hbm.at[pl.ds(base, b_per_w)])

    return pl.kernel(
        body,
        mesh=mesh,
        out_type=jax.ShapeDtypeStruct((batch, dim), jnp.float32),
        scratch_types=[
            pltpu.VMEM((b_per_w,), jnp.int32),
            pltpu.VMEM((b_per_w, dim), jnp.float32),
            pltpu.SemaphoreType.DMA,
        ],
        compiler_params=pltpu.CompilerParams(
            use_tc_tiling_on_sc=False,
            skip_device_barrier=True,
        ),
    )


@jax.jit
def kernel(indices, table):
    batch = indices.shape[0]
    dim = table.shape[1]
    idx = indices.astype(jnp.int32)
    return _make_gather(batch, dim)(table, idx)
